# Initial kernel scaffold; baseline (speedup 1.0000x reference)
import jax
import jax.numpy as jnp
from jax.experimental import pallas as pl


def kernel(node_embeddings, edge_index, W_rel, Wu1, bu1, Wu2, bu2, ln_g, ln_b):
    raise NotImplementedError("write your pallas kernel here")


# speedup vs baseline: 7.7759x; 7.7759x over previous
import functools

import jax
import jax.numpy as jnp
from jax import lax
from jax.experimental import pallas as pl
from jax.experimental.pallas import tpu as pltpu
from jax.experimental.pallas import tpu_sc as plsc

N = 10000
E = 320000
D = 128
NUM_LAYERS = 3
EPS = 1e-5

NC = 2
NS = 16
NW = NC * NS
N_PAD = 10240
K = 125
C = E // (NW * K)

_mesh = plsc.VectorSubcoreMesh(
    core_axis_name="c", subcore_axis_name="s", num_cores=NC, num_subcores=NS
)


@functools.partial(
    pl.kernel,
    out_type=jax.ShapeDtypeStruct((NC, N_PAD, D), jnp.float32),
    mesh=_mesh,
    scratch_types=[
        pltpu.VMEM((C, K), jnp.int32),
        pltpu.VMEM((C, K), jnp.int32),
        pltpu.VMEM((K, D), jnp.float32),
        pltpu.VMEM((16, D), jnp.float32),
        pltpu.VMEM_SHARED((N_PAD, D), jnp.float32),
    ],
)
def _sc_segsum(t_hbm, src_hbm, dst_hbm, out_hbm, src_v, dst_v, rows_v, zb, y_sh):
    c = lax.axis_index("c")
    s = lax.axis_index("s")
    pltpu.sync_copy(src_hbm.at[c, s], src_v)
    pltpu.sync_copy(dst_hbm.at[c, s], dst_v)
    for r in range(16):
        for q in range(D // 16):
            zb[r, pl.ds(q * 16, 16)] = jnp.zeros((16,), jnp.float32)
    rows_per_tile = N_PAD // NS
    base = s * rows_per_tile

    def zbody(b, carry):
        pltpu.sync_copy(zb, y_sh.at[pl.ds(base + b * 16, 16)])
        return carry

    lax.fori_loop(0, rows_per_tile // 16, zbody, 0)
    plsc.subcore_barrier()

    def ebody(j, carry):
        pltpu.sync_copy(t_hbm.at[src_v.at[j]], rows_v)
        pltpu.sync_copy(rows_v, y_sh.at[dst_v.at[j]], add=True)
        return carry

    lax.fori_loop(0, C, ebody, 0)
    plsc.subcore_barrier()
    pltpu.sync_copy(
        y_sh.at[pl.ds(base, rows_per_tile)],
        out_hbm.at[c, pl.ds(base, rows_per_tile)],
    )


@functools.partial(
    pl.kernel,
    out_type=jax.ShapeDtypeStruct((NC, N_PAD, D), jnp.float32),
    mesh=_mesh,
    scratch_types=[
        pltpu.VMEM((C, K), jnp.int32),
        pltpu.VMEM((K, D), jnp.float32),
        pltpu.VMEM((16, D), jnp.float32),
        pltpu.VMEM_SHARED((N_PAD, D), jnp.float32),
    ],
)
def _sc_deg(dst_hbm, out_hbm, dst_v, ones_v, zb, deg_sh):
    c = lax.axis_index("c")
    s = lax.axis_index("s")
    pltpu.sync_copy(dst_hbm.at[c, s], dst_v)
    for r in range(K):
        for q in range(D // 16):
            ones_v[r, pl.ds(q * 16, 16)] = jnp.ones((16,), jnp.float32)
    for r in range(16):
        for q in range(D // 16):
            zb[r, pl.ds(q * 16, 16)] = jnp.zeros((16,), jnp.float32)
    rows_per_tile = N_PAD // NS
    base = s * rows_per_tile

    def zbody(b, carry):
        pltpu.sync_copy(zb, deg_sh.at[pl.ds(base + b * 16, 16)])
        return carry

    lax.fori_loop(0, rows_per_tile // 16, zbody, 0)
    plsc.subcore_barrier()

    def ebody(j, carry):
        pltpu.sync_copy(ones_v, deg_sh.at[dst_v.at[j]], add=True)
        return carry

    lax.fori_loop(0, C, ebody, 0)
    plsc.subcore_barrier()
    pltpu.sync_copy(
        deg_sh.at[pl.ds(base, rows_per_tile)],
        out_hbm.at[c, pl.ds(base, rows_per_tile)],
    )


BN = 1024


def _mm_body(x_ref, w_ref, o_ref):
    o_ref[...] = jnp.dot(x_ref[...], w_ref[...], preferred_element_type=jnp.float32)


_tc_matmul = pl.pallas_call(
    _mm_body,
    grid=(N_PAD // BN,),
    in_specs=[
        pl.BlockSpec((BN, D), lambda i: (i, 0)),
        pl.BlockSpec((D, D), lambda i: (0, 0)),
    ],
    out_specs=pl.BlockSpec((BN, D), lambda i: (i, 0)),
    out_shape=jax.ShapeDtypeStruct((N_PAD, D), jnp.float32),
)


def _upd_body(x_ref, y_ref, dg_ref, wu1_ref, bu1_ref, wu2_ref, bu2_ref,
              g_ref, b_ref, wr_ref, xo_ref, to_ref):
    x = x_ref[...]
    y = y_ref[0] + y_ref[1]
    deg = jnp.maximum(dg_ref[0, :, 0:1] + dg_ref[1, :, 0:1], 1.0)
    agg = y / deg
    u = jnp.concatenate([x, agg], axis=1)
    h = jnp.maximum(
        jnp.dot(u, wu1_ref[...], preferred_element_type=jnp.float32) + bu1_ref[...],
        0.0,
    )
    upd = jnp.dot(h, wu2_ref[...], preferred_element_type=jnp.float32) + bu2_ref[...]
    mu = jnp.mean(upd, axis=-1, keepdims=True)
    var = jnp.mean((upd - mu) ** 2, axis=-1, keepdims=True)
    upd = (upd - mu) * lax.rsqrt(var + EPS) * g_ref[...] + b_ref[...]
    xn = x + upd
    xo_ref[...] = xn
    to_ref[...] = jnp.dot(xn, wr_ref[...], preferred_element_type=jnp.float32)


_tc_update = pl.pallas_call(
    _upd_body,
    grid=(N_PAD // BN,),
    in_specs=[
        pl.BlockSpec((BN, D), lambda i: (i, 0)),
        pl.BlockSpec((NC, BN, D), lambda i: (0, i, 0)),
        pl.BlockSpec((NC, BN, D), lambda i: (0, i, 0)),
        pl.BlockSpec((2 * D, D), lambda i: (0, 0)),
        pl.BlockSpec((1, D), lambda i: (0, 0)),
        pl.BlockSpec((D, D), lambda i: (0, 0)),
        pl.BlockSpec((1, D), lambda i: (0, 0)),
        pl.BlockSpec((1, D), lambda i: (0, 0)),
        pl.BlockSpec((1, D), lambda i: (0, 0)),
        pl.BlockSpec((D, D), lambda i: (0, 0)),
    ],
    out_specs=[
        pl.BlockSpec((BN, D), lambda i: (i, 0)),
        pl.BlockSpec((BN, D), lambda i: (i, 0)),
    ],
    out_shape=[
        jax.ShapeDtypeStruct((N_PAD, D), jnp.float32),
        jax.ShapeDtypeStruct((N_PAD, D), jnp.float32),
    ],
)


def kernel(node_embeddings, edge_index, W_rel, Wu1, bu1, Wu2, bu2, ln_g, ln_b):
    x = jnp.zeros((N_PAD, D), jnp.float32).at[:N].set(node_embeddings)
    src_r = edge_index[0].reshape(NC, NS, C, K)
    dst_r = edge_index[1].reshape(NC, NS, C, K)
    degp = _sc_deg(dst_r)
    t = _tc_matmul(x, W_rel)
    b1 = bu1.reshape(1, D)
    b2 = bu2.reshape(1, D)
    g = ln_g.reshape(1, D)
    b = ln_b.reshape(1, D)
    for _ in range(NUM_LAYERS):
        y = _sc_segsum(t, src_r, dst_r)
        x, t = _tc_update(x, y, degp, Wu1, b1, Wu2, b2, g, b, W_rel)
    return x[:N]


# speedup vs baseline: 10.8400x; 1.3941x over previous
import functools

import jax
import jax.numpy as jnp
from jax import lax
from jax.experimental import pallas as pl
from jax.experimental.pallas import tpu as pltpu
from jax.experimental.pallas import tpu_sc as plsc

N = 10000
E = 320000
D = 128
NUM_LAYERS = 3
EPS = 1e-5

NC = 2
NS = 16
NW = NC * NS
N_PAD = 10240
K = 125
C = E // (NW * K)

_mesh = plsc.VectorSubcoreMesh(
    core_axis_name="c", subcore_axis_name="s", num_cores=NC, num_subcores=NS
)


@functools.partial(
    pl.kernel,
    out_type=jax.ShapeDtypeStruct((NC, N_PAD, D), jnp.float32),
    mesh=_mesh,
    scratch_types=[
        pltpu.VMEM((C // 2, K), jnp.int32),
        pltpu.VMEM((C // 2, K), jnp.int32),
        pltpu.VMEM((K, D), jnp.float32),
        pltpu.VMEM((K, D), jnp.float32),
        pltpu.VMEM_SHARED((N_PAD, D), jnp.float32),
        pltpu.SemaphoreType.DMA,
        pltpu.SemaphoreType.DMA,
    ],
)
def _sc_segsum(t_hbm, src_hbm, dst_hbm, out_hbm, src_v, dst_v, rows_a, rows_b,
               y_sh, sem_a, sem_b):
    c = lax.axis_index("c")
    s = lax.axis_index("s")
    for r in range(16):
        for q in range(D // 16):
            rows_a[r, pl.ds(q * 16, 16)] = jnp.zeros((16,), jnp.float32)
    rows_per_tile = N_PAD // NS
    base = s * rows_per_tile

    def zbody(b, carry):
        pltpu.sync_copy(rows_a.at[pl.ds(0, 16)], y_sh.at[pl.ds(base + b * 16, 16)])
        return carry

    lax.fori_loop(0, rows_per_tile // 16, zbody, 0)
    plsc.subcore_barrier()

    C2 = C // 2
    for h in range(2):
        pltpu.sync_copy(src_hbm.at[c, s, pl.ds(h * C2, C2)], src_v)
        pltpu.sync_copy(dst_hbm.at[c, s, pl.ds(h * C2, C2)], dst_v)
        pltpu.async_copy(t_hbm.at[src_v.at[0]], rows_a, sem_a)

        def ebody(i, carry):
            j = 2 * i
            pltpu.async_copy(t_hbm.at[src_v.at[j + 1]], rows_b, sem_b)
            pltpu.make_async_copy(t_hbm.at[src_v.at[j]], rows_a, sem_a).wait()
            pltpu.sync_copy(rows_a, y_sh.at[dst_v.at[j]], add=True)

            @pl.when(j + 2 < C2)
            def _():
                pltpu.async_copy(t_hbm.at[src_v.at[j + 2]], rows_a, sem_a)

            pltpu.make_async_copy(t_hbm.at[src_v.at[j + 1]], rows_b, sem_b).wait()
            pltpu.sync_copy(rows_b, y_sh.at[dst_v.at[j + 1]], add=True)
            return carry

        lax.fori_loop(0, C2 // 2, ebody, 0)
    plsc.subcore_barrier()
    pltpu.sync_copy(
        y_sh.at[pl.ds(base, rows_per_tile)],
        out_hbm.at[c, pl.ds(base, rows_per_tile)],
    )


@functools.partial(
    pl.kernel,
    out_type=jax.ShapeDtypeStruct((NC, N_PAD, D), jnp.float32),
    mesh=_mesh,
    scratch_types=[
        pltpu.VMEM((C, K), jnp.int32),
        pltpu.VMEM((K, D), jnp.float32),
        pltpu.VMEM((16, D), jnp.float32),
        pltpu.VMEM_SHARED((N_PAD, D), jnp.float32),
    ],
)
def _sc_deg(dst_hbm, out_hbm, dst_v, ones_v, zb, deg_sh):
    c = lax.axis_index("c")
    s = lax.axis_index("s")
    pltpu.sync_copy(dst_hbm.at[c, s], dst_v)
    for r in range(K):
        for q in range(D // 16):
            ones_v[r, pl.ds(q * 16, 16)] = jnp.ones((16,), jnp.float32)
    for r in range(16):
        for q in range(D // 16):
            zb[r, pl.ds(q * 16, 16)] = jnp.zeros((16,), jnp.float32)
    rows_per_tile = N_PAD // NS
    base = s * rows_per_tile

    def zbody(b, carry):
        pltpu.sync_copy(zb, deg_sh.at[pl.ds(base + b * 16, 16)])
        return carry

    lax.fori_loop(0, rows_per_tile // 16, zbody, 0)
    plsc.subcore_barrier()

    def ebody(j, carry):
        pltpu.sync_copy(ones_v, deg_sh.at[dst_v.at[j]], add=True)
        return carry

    lax.fori_loop(0, C, ebody, 0)
    plsc.subcore_barrier()
    pltpu.sync_copy(
        deg_sh.at[pl.ds(base, rows_per_tile)],
        out_hbm.at[c, pl.ds(base, rows_per_tile)],
    )


BN = 1024


def _mm_body(x_ref, w_ref, o_ref):
    o_ref[...] = jnp.dot(x_ref[...], w_ref[...], preferred_element_type=jnp.float32)


_tc_matmul = pl.pallas_call(
    _mm_body,
    grid=(N_PAD // BN,),
    in_specs=[
        pl.BlockSpec((BN, D), lambda i: (i, 0)),
        pl.BlockSpec((D, D), lambda i: (0, 0)),
    ],
    out_specs=pl.BlockSpec((BN, D), lambda i: (i, 0)),
    out_shape=jax.ShapeDtypeStruct((N_PAD, D), jnp.float32),
)


def _upd_body(x_ref, y_ref, dg_ref, wu1_ref, bu1_ref, wu2_ref, bu2_ref,
              g_ref, b_ref, wr_ref, xo_ref, to_ref):
    x = x_ref[...]
    y = y_ref[0] + y_ref[1]
    deg = jnp.maximum(dg_ref[0, :, 0:1] + dg_ref[1, :, 0:1], 1.0)
    agg = y / deg
    u = jnp.concatenate([x, agg], axis=1)
    h = jnp.maximum(
        jnp.dot(u, wu1_ref[...], preferred_element_type=jnp.float32) + bu1_ref[...],
        0.0,
    )
    upd = jnp.dot(h, wu2_ref[...], preferred_element_type=jnp.float32) + bu2_ref[...]
    mu = jnp.mean(upd, axis=-1, keepdims=True)
    var = jnp.mean((upd - mu) ** 2, axis=-1, keepdims=True)
    upd = (upd - mu) * lax.rsqrt(var + EPS) * g_ref[...] + b_ref[...]
    xn = x + upd
    xo_ref[...] = xn
    to_ref[...] = jnp.dot(xn, wr_ref[...], preferred_element_type=jnp.float32)


_tc_update = pl.pallas_call(
    _upd_body,
    grid=(N_PAD // BN,),
    in_specs=[
        pl.BlockSpec((BN, D), lambda i: (i, 0)),
        pl.BlockSpec((NC, BN, D), lambda i: (0, i, 0)),
        pl.BlockSpec((NC, BN, D), lambda i: (0, i, 0)),
        pl.BlockSpec((2 * D, D), lambda i: (0, 0)),
        pl.BlockSpec((1, D), lambda i: (0, 0)),
        pl.BlockSpec((D, D), lambda i: (0, 0)),
        pl.BlockSpec((1, D), lambda i: (0, 0)),
        pl.BlockSpec((1, D), lambda i: (0, 0)),
        pl.BlockSpec((1, D), lambda i: (0, 0)),
        pl.BlockSpec((D, D), lambda i: (0, 0)),
    ],
    out_specs=[
        pl.BlockSpec((BN, D), lambda i: (i, 0)),
        pl.BlockSpec((BN, D), lambda i: (i, 0)),
    ],
    out_shape=[
        jax.ShapeDtypeStruct((N_PAD, D), jnp.float32),
        jax.ShapeDtypeStruct((N_PAD, D), jnp.float32),
    ],
)


def kernel(node_embeddings, edge_index, W_rel, Wu1, bu1, Wu2, bu2, ln_g, ln_b):
    x = jnp.zeros((N_PAD, D), jnp.float32).at[:N].set(node_embeddings)
    src_r = edge_index[0].reshape(NC, NS, C, K)
    dst_r = edge_index[1].reshape(NC, NS, C, K)
    degp = _sc_deg(dst_r)
    t = _tc_matmul(x, W_rel)
    b1 = bu1.reshape(1, D)
    b2 = bu2.reshape(1, D)
    g = ln_g.reshape(1, D)
    b = ln_b.reshape(1, D)
    for _ in range(NUM_LAYERS):
        y = _sc_segsum(t, src_r, dst_r)
        x, t = _tc_update(x, y, degp, Wu1, b1, Wu2, b2, g, b, W_rel)
    return x[:N]


# speedup vs baseline: 10.8713x; 1.0029x over previous
import functools

import jax
import jax.numpy as jnp
from jax import lax
from jax.experimental import pallas as pl
from jax.experimental.pallas import tpu as pltpu
from jax.experimental.pallas import tpu_sc as plsc

N = 10000
E = 320000
D = 128
NUM_LAYERS = 3
EPS = 1e-5

NC = 2
NS = 16
NW = NC * NS
N_PAD = 10240
K = 125
C = E // (NW * K)

_mesh = plsc.VectorSubcoreMesh(
    core_axis_name="c", subcore_axis_name="s", num_cores=NC, num_subcores=NS
)


@functools.partial(
    pl.kernel,
    out_type=jax.ShapeDtypeStruct((NC, N_PAD, D), jnp.float32),
    mesh=_mesh,
    scratch_types=[
        pltpu.VMEM((C // 2, K), jnp.int32),
        pltpu.VMEM((C // 2, K), jnp.int32),
        pltpu.VMEM((K, D), jnp.float32),
        pltpu.VMEM((K, D), jnp.float32),
        pltpu.VMEM_SHARED((N_PAD, D), jnp.float32),
        pltpu.SemaphoreType.DMA,
        pltpu.SemaphoreType.DMA,
    ],
)
def _sc_segsum(t_hbm, src_hbm, dst_hbm, out_hbm, src_v, dst_v, rows_a, rows_b,
               y_sh, sem_a, sem_b):
    c = lax.axis_index("c")
    s = lax.axis_index("s")
    for r in range(16):
        for q in range(D // 16):
            rows_a[r, pl.ds(q * 16, 16)] = jnp.zeros((16,), jnp.float32)
    rows_per_tile = N_PAD // NS
    base = s * rows_per_tile

    def zbody(b, carry):
        pltpu.sync_copy(rows_a.at[pl.ds(0, 16)], y_sh.at[pl.ds(base + b * 16, 16)])
        return carry

    lax.fori_loop(0, rows_per_tile // 16, zbody, 0)
    plsc.subcore_barrier()

    C2 = C // 2
    for h in range(2):
        pltpu.sync_copy(src_hbm.at[c, s, pl.ds(h * C2, C2)], src_v)
        pltpu.sync_copy(dst_hbm.at[c, s, pl.ds(h * C2, C2)], dst_v)
        pltpu.async_copy(t_hbm.at[src_v.at[0]], rows_a, sem_a)

        def ebody(i, carry):
            j = 2 * i
            pltpu.async_copy(t_hbm.at[src_v.at[j + 1]], rows_b, sem_b)
            pltpu.make_async_copy(t_hbm.at[src_v.at[j]], rows_a, sem_a).wait()
            pltpu.sync_copy(rows_a, y_sh.at[dst_v.at[j]], add=True)

            @pl.when(j + 2 < C2)
            def _():
                pltpu.async_copy(t_hbm.at[src_v.at[j + 2]], rows_a, sem_a)

            pltpu.make_async_copy(t_hbm.at[src_v.at[j + 1]], rows_b, sem_b).wait()
            pltpu.sync_copy(rows_b, y_sh.at[dst_v.at[j + 1]], add=True)
            return carry

        lax.fori_loop(0, C2 // 2, ebody, 0)
    plsc.subcore_barrier()
    pltpu.sync_copy(
        y_sh.at[pl.ds(base, rows_per_tile)],
        out_hbm.at[c, pl.ds(base, rows_per_tile)],
    )


DW = 128


@functools.partial(
    pl.kernel,
    out_type=jax.ShapeDtypeStruct((NC, N_PAD, DW), jnp.float32),
    mesh=_mesh,
    scratch_types=[
        pltpu.VMEM((C, K), jnp.int32),
        pltpu.VMEM((K, DW), jnp.float32),
        pltpu.VMEM((16, DW), jnp.float32),
        pltpu.VMEM_SHARED((N_PAD, DW), jnp.float32),
    ],
)
def _sc_deg(dst_hbm, out_hbm, dst_v, ones_v, zb, deg_sh):
    c = lax.axis_index("c")
    s = lax.axis_index("s")
    pltpu.sync_copy(dst_hbm.at[c, s], dst_v)
    for r in range(K):
        for q in range(DW // 16):
            ones_v[r, pl.ds(q * 16, 16)] = jnp.ones((16,), jnp.float32)
    for r in range(16):
        for q in range(DW // 16):
            zb[r, pl.ds(q * 16, 16)] = jnp.zeros((16,), jnp.float32)
    rows_per_tile = N_PAD // NS
    base = s * rows_per_tile

    def zbody(b, carry):
        pltpu.sync_copy(zb, deg_sh.at[pl.ds(base + b * 16, 16)])
        return carry

    lax.fori_loop(0, rows_per_tile // 16, zbody, 0)
    plsc.subcore_barrier()

    def ebody(j, carry):
        pltpu.sync_copy(ones_v, deg_sh.at[dst_v.at[j]], add=True)
        return carry

    lax.fori_loop(0, C, ebody, 0)
    plsc.subcore_barrier()
    pltpu.sync_copy(
        deg_sh.at[pl.ds(base, rows_per_tile)],
        out_hbm.at[c, pl.ds(base, rows_per_tile)],
    )


BN = 1024


def _mm_body(x_ref, w_ref, o_ref):
    o_ref[...] = jnp.dot(x_ref[...], w_ref[...], preferred_element_type=jnp.float32)


_tc_matmul = pl.pallas_call(
    _mm_body,
    grid=(N_PAD // BN,),
    in_specs=[
        pl.BlockSpec((BN, D), lambda i: (i, 0)),
        pl.BlockSpec((D, D), lambda i: (0, 0)),
    ],
    out_specs=pl.BlockSpec((BN, D), lambda i: (i, 0)),
    out_shape=jax.ShapeDtypeStruct((N_PAD, D), jnp.float32),
)


def _update_rows(x_ref, y_ref, dg_ref, wu1_ref, bu1_ref, wu2_ref, bu2_ref,
                 g_ref, b_ref):
    x = x_ref[...]
    y = y_ref[0] + y_ref[1]
    deg = jnp.maximum(dg_ref[0, :, 0:1] + dg_ref[1, :, 0:1], 1.0)
    agg = y / deg
    u = jnp.concatenate([x, agg], axis=1)
    h = jnp.maximum(
        jnp.dot(u, wu1_ref[...], preferred_element_type=jnp.float32) + bu1_ref[...],
        0.0,
    )
    upd = jnp.dot(h, wu2_ref[...], preferred_element_type=jnp.float32) + bu2_ref[...]
    mu = jnp.mean(upd, axis=-1, keepdims=True)
    var = jnp.mean((upd - mu) ** 2, axis=-1, keepdims=True)
    upd = (upd - mu) * lax.rsqrt(var + EPS) * g_ref[...] + b_ref[...]
    return x + upd


def _upd_body(x_ref, y_ref, dg_ref, wu1_ref, bu1_ref, wu2_ref, bu2_ref,
              g_ref, b_ref, wr_ref, xo_ref, to_ref):
    xn = _update_rows(x_ref, y_ref, dg_ref, wu1_ref, bu1_ref, wu2_ref, bu2_ref,
                      g_ref, b_ref)
    xo_ref[...] = xn
    to_ref[...] = jnp.dot(xn, wr_ref[...], preferred_element_type=jnp.float32)


def _upd_last_body(x_ref, y_ref, dg_ref, wu1_ref, bu1_ref, wu2_ref, bu2_ref,
                   g_ref, b_ref, xo_ref):
    xo_ref[...] = _update_rows(x_ref, y_ref, dg_ref, wu1_ref, bu1_ref, wu2_ref,
                               bu2_ref, g_ref, b_ref)


_upd_in_specs = [
    pl.BlockSpec((BN, D), lambda i: (i, 0)),
    pl.BlockSpec((NC, BN, D), lambda i: (0, i, 0)),
    pl.BlockSpec((NC, BN, DW), lambda i: (0, i, 0)),
    pl.BlockSpec((2 * D, D), lambda i: (0, 0)),
    pl.BlockSpec((1, D), lambda i: (0, 0)),
    pl.BlockSpec((D, D), lambda i: (0, 0)),
    pl.BlockSpec((1, D), lambda i: (0, 0)),
    pl.BlockSpec((1, D), lambda i: (0, 0)),
    pl.BlockSpec((1, D), lambda i: (0, 0)),
]

_tc_update = pl.pallas_call(
    _upd_body,
    grid=(N_PAD // BN,),
    in_specs=_upd_in_specs + [pl.BlockSpec((D, D), lambda i: (0, 0))],
    out_specs=[
        pl.BlockSpec((BN, D), lambda i: (i, 0)),
        pl.BlockSpec((BN, D), lambda i: (i, 0)),
    ],
    out_shape=[
        jax.ShapeDtypeStruct((N_PAD, D), jnp.float32),
        jax.ShapeDtypeStruct((N_PAD, D), jnp.float32),
    ],
)

_tc_update_last = pl.pallas_call(
    _upd_last_body,
    grid=(N_PAD // BN,),
    in_specs=_upd_in_specs,
    out_specs=pl.BlockSpec((BN, D), lambda i: (i, 0)),
    out_shape=jax.ShapeDtypeStruct((N_PAD, D), jnp.float32),
)


def kernel(node_embeddings, edge_index, W_rel, Wu1, bu1, Wu2, bu2, ln_g, ln_b):
    x = jnp.zeros((N_PAD, D), jnp.float32).at[:N].set(node_embeddings)
    src_r = edge_index[0].reshape(NC, NS, C, K)
    dst_r = edge_index[1].reshape(NC, NS, C, K)
    degp = _sc_deg(dst_r)
    t = _tc_matmul(x, W_rel)
    b1 = bu1.reshape(1, D)
    b2 = bu2.reshape(1, D)
    g = ln_g.reshape(1, D)
    b = ln_b.reshape(1, D)
    for _ in range(NUM_LAYERS - 1):
        y = _sc_segsum(t, src_r, dst_r)
        x, t = _tc_update(x, y, degp, Wu1, b1, Wu2, b2, g, b, W_rel)
    y = _sc_segsum(t, src_r, dst_r)
    x = _tc_update_last(x, y, degp, Wu1, b1, Wu2, b2, g, b)
    return x[:N]


# speedup vs baseline: 10.9177x; 1.0043x over previous
import functools

import jax
import jax.numpy as jnp
from jax import lax
from jax.experimental import pallas as pl
from jax.experimental.pallas import tpu as pltpu
from jax.experimental.pallas import tpu_sc as plsc

N = 10000
E = 320000
D = 128
NUM_LAYERS = 3
EPS = 1e-5

NC = 2
NS = 16
NW = NC * NS
N_PAD = 10240
K = 125
C = E // (NW * K)

_mesh = plsc.VectorSubcoreMesh(
    core_axis_name="c", subcore_axis_name="s", num_cores=NC, num_subcores=NS
)


@functools.partial(
    pl.kernel,
    out_type=jax.ShapeDtypeStruct((NC, N_PAD, D), jnp.float32),
    mesh=_mesh,
    scratch_types=[
        pltpu.VMEM((C // 2, K), jnp.int32),
        pltpu.VMEM((C // 2, K), jnp.int32),
        pltpu.VMEM((K, D), jnp.float32),
        pltpu.VMEM((K, D), jnp.float32),
        pltpu.VMEM_SHARED((N_PAD, D), jnp.float32),
        pltpu.SemaphoreType.DMA,
        pltpu.SemaphoreType.DMA,
    ],
)
def _sc_segsum(t_hbm, src_hbm, dst_hbm, out_hbm, src_v, dst_v, rows_a, rows_b,
               y_sh, sem_a, sem_b):
    c = lax.axis_index("c")
    s = lax.axis_index("s")
    for r in range(80):
        for q in range(D // 16):
            rows_a[r, pl.ds(q * 16, 16)] = jnp.zeros((16,), jnp.float32)
    rows_per_tile = N_PAD // NS
    base = s * rows_per_tile

    def zbody(b, carry):
        pltpu.sync_copy(rows_a.at[pl.ds(0, 80)], y_sh.at[pl.ds(base + b * 80, 80)])
        return carry

    lax.fori_loop(0, rows_per_tile // 80, zbody, 0)
    plsc.subcore_barrier()

    C2 = C // 2
    for h in range(2):
        pltpu.sync_copy(src_hbm.at[c, s, pl.ds(h * C2, C2)], src_v)
        pltpu.sync_copy(dst_hbm.at[c, s, pl.ds(h * C2, C2)], dst_v)
        pltpu.async_copy(t_hbm.at[src_v.at[0]], rows_a, sem_a)

        def ebody(i, carry):
            j = 2 * i
            pltpu.async_copy(t_hbm.at[src_v.at[j + 1]], rows_b, sem_b)
            pltpu.make_async_copy(t_hbm.at[src_v.at[j]], rows_a, sem_a).wait()
            pltpu.sync_copy(rows_a, y_sh.at[dst_v.at[j]], add=True)

            @pl.when(j + 2 < C2)
            def _():
                pltpu.async_copy(t_hbm.at[src_v.at[j + 2]], rows_a, sem_a)

            pltpu.make_async_copy(t_hbm.at[src_v.at[j + 1]], rows_b, sem_b).wait()
            pltpu.sync_copy(rows_b, y_sh.at[dst_v.at[j + 1]], add=True)
            return carry

        lax.fori_loop(0, C2 // 2, ebody, 0)
    plsc.subcore_barrier()
    pltpu.sync_copy(
        y_sh.at[pl.ds(base, rows_per_tile)],
        out_hbm.at[c, pl.ds(base, rows_per_tile)],
    )


DW = 128


@functools.partial(
    pl.kernel,
    out_type=jax.ShapeDtypeStruct((NC, N_PAD, DW), jnp.float32),
    mesh=_mesh,
    scratch_types=[
        pltpu.VMEM((C, K), jnp.int32),
        pltpu.VMEM((K, DW), jnp.float32),
        pltpu.VMEM((80, DW), jnp.float32),
        pltpu.VMEM_SHARED((N_PAD, DW), jnp.float32),
    ],
)
def _sc_deg(dst_hbm, out_hbm, dst_v, ones_v, zb, deg_sh):
    c = lax.axis_index("c")
    s = lax.axis_index("s")
    pltpu.sync_copy(dst_hbm.at[c, s], dst_v)
    for r in range(K):
        for q in range(DW // 16):
            ones_v[r, pl.ds(q * 16, 16)] = jnp.ones((16,), jnp.float32)
    for r in range(80):
        for q in range(DW // 16):
            zb[r, pl.ds(q * 16, 16)] = jnp.zeros((16,), jnp.float32)
    rows_per_tile = N_PAD // NS
    base = s * rows_per_tile

    def zbody(b, carry):
        pltpu.sync_copy(zb, deg_sh.at[pl.ds(base + b * 80, 80)])
        return carry

    lax.fori_loop(0, rows_per_tile // 80, zbody, 0)
    plsc.subcore_barrier()

    def ebody(j, carry):
        pltpu.sync_copy(ones_v, deg_sh.at[dst_v.at[j]], add=True)
        return carry

    lax.fori_loop(0, C, ebody, 0)
    plsc.subcore_barrier()
    pltpu.sync_copy(
        deg_sh.at[pl.ds(base, rows_per_tile)],
        out_hbm.at[c, pl.ds(base, rows_per_tile)],
    )


BN = 1024


def _mm_body(x_ref, w_ref, o_ref):
    o_ref[...] = jnp.dot(x_ref[...], w_ref[...], preferred_element_type=jnp.float32)


_tc_matmul = pl.pallas_call(
    _mm_body,
    grid=(N_PAD // BN,),
    in_specs=[
        pl.BlockSpec((BN, D), lambda i: (i, 0)),
        pl.BlockSpec((D, D), lambda i: (0, 0)),
    ],
    out_specs=pl.BlockSpec((BN, D), lambda i: (i, 0)),
    out_shape=jax.ShapeDtypeStruct((N_PAD, D), jnp.float32),
)


def _update_rows(x_ref, y_ref, dg_ref, wu1_ref, bu1_ref, wu2_ref, bu2_ref,
                 g_ref, b_ref):
    x = x_ref[...]
    y = y_ref[0] + y_ref[1]
    deg = jnp.maximum(dg_ref[0, :, 0:1] + dg_ref[1, :, 0:1], 1.0)
    agg = y / deg
    u = jnp.concatenate([x, agg], axis=1)
    h = jnp.maximum(
        jnp.dot(u, wu1_ref[...], preferred_element_type=jnp.float32) + bu1_ref[...],
        0.0,
    )
    upd = jnp.dot(h, wu2_ref[...], preferred_element_type=jnp.float32) + bu2_ref[...]
    mu = jnp.mean(upd, axis=-1, keepdims=True)
    var = jnp.mean((upd - mu) ** 2, axis=-1, keepdims=True)
    upd = (upd - mu) * lax.rsqrt(var + EPS) * g_ref[...] + b_ref[...]
    return x + upd


def _upd_body(x_ref, y_ref, dg_ref, wu1_ref, bu1_ref, wu2_ref, bu2_ref,
              g_ref, b_ref, wr_ref, xo_ref, to_ref):
    xn = _update_rows(x_ref, y_ref, dg_ref, wu1_ref, bu1_ref, wu2_ref, bu2_ref,
                      g_ref, b_ref)
    xo_ref[...] = xn
    to_ref[...] = jnp.dot(xn, wr_ref[...], preferred_element_type=jnp.float32)


def _upd_last_body(x_ref, y_ref, dg_ref, wu1_ref, bu1_ref, wu2_ref, bu2_ref,
                   g_ref, b_ref, xo_ref):
    xo_ref[...] = _update_rows(x_ref, y_ref, dg_ref, wu1_ref, bu1_ref, wu2_ref,
                               bu2_ref, g_ref, b_ref)


_upd_in_specs = [
    pl.BlockSpec((BN, D), lambda i: (i, 0)),
    pl.BlockSpec((NC, BN, D), lambda i: (0, i, 0)),
    pl.BlockSpec((NC, BN, DW), lambda i: (0, i, 0)),
    pl.BlockSpec((2 * D, D), lambda i: (0, 0)),
    pl.BlockSpec((1, D), lambda i: (0, 0)),
    pl.BlockSpec((D, D), lambda i: (0, 0)),
    pl.BlockSpec((1, D), lambda i: (0, 0)),
    pl.BlockSpec((1, D), lambda i: (0, 0)),
    pl.BlockSpec((1, D), lambda i: (0, 0)),
]

_tc_update = pl.pallas_call(
    _upd_body,
    grid=(N_PAD // BN,),
    in_specs=_upd_in_specs + [pl.BlockSpec((D, D), lambda i: (0, 0))],
    out_specs=[
        pl.BlockSpec((BN, D), lambda i: (i, 0)),
        pl.BlockSpec((BN, D), lambda i: (i, 0)),
    ],
    out_shape=[
        jax.ShapeDtypeStruct((N_PAD, D), jnp.float32),
        jax.ShapeDtypeStruct((N_PAD, D), jnp.float32),
    ],
)

_tc_update_last = pl.pallas_call(
    _upd_last_body,
    grid=(N_PAD // BN,),
    in_specs=_upd_in_specs,
    out_specs=pl.BlockSpec((BN, D), lambda i: (i, 0)),
    out_shape=jax.ShapeDtypeStruct((N_PAD, D), jnp.float32),
)


def kernel(node_embeddings, edge_index, W_rel, Wu1, bu1, Wu2, bu2, ln_g, ln_b):
    x = jnp.zeros((N_PAD, D), jnp.float32).at[:N].set(node_embeddings)
    src_r = edge_index[0].reshape(NC, NS, C, K)
    dst_r = edge_index[1].reshape(NC, NS, C, K)
    degp = _sc_deg(dst_r)
    t = _tc_matmul(x, W_rel)
    b1 = bu1.reshape(1, D)
    b2 = bu2.reshape(1, D)
    g = ln_g.reshape(1, D)
    b = ln_b.reshape(1, D)
    for _ in range(NUM_LAYERS - 1):
        y = _sc_segsum(t, src_r, dst_r)
        x, t = _tc_update(x, y, degp, Wu1, b1, Wu2, b2, g, b, W_rel)
    y = _sc_segsum(t, src_r, dst_r)
    x = _tc_update_last(x, y, degp, Wu1, b1, Wu2, b2, g, b)
    return x[:N]


# speedup vs baseline: 11.0239x; 1.0097x over previous
import functools

import jax
import jax.numpy as jnp
from jax import lax
from jax.experimental import pallas as pl
from jax.experimental.pallas import tpu as pltpu
from jax.experimental.pallas import tpu_sc as plsc

N = 10000
E = 320000
D = 128
NUM_LAYERS = 3
EPS = 1e-5

NC = 2
NS = 16
NW = NC * NS
N_PAD = 10240
K = 125
C = E // (NW * K)

_mesh = plsc.VectorSubcoreMesh(
    core_axis_name="c", subcore_axis_name="s", num_cores=NC, num_subcores=NS
)


@functools.partial(
    pl.kernel,
    out_type=jax.ShapeDtypeStruct((NC, N_PAD, D), jnp.float32),
    mesh=_mesh,
    scratch_types=[
        pltpu.VMEM((C // 2, K), jnp.int32),
        pltpu.VMEM((C // 2, K), jnp.int32),
        pltpu.VMEM((K, D), jnp.float32),
        pltpu.VMEM((K, D), jnp.float32),
        pltpu.VMEM_SHARED((N_PAD, D), jnp.float32),
        pltpu.SemaphoreType.DMA,
        pltpu.SemaphoreType.DMA,
    ],
)
def _sc_segsum(t_hbm, src_hbm, dst_hbm, out_hbm, src_v, dst_v, rows_a, rows_b,
               y_sh, sem_a, sem_b):
    c = lax.axis_index("c")
    s = lax.axis_index("s")
    for r in range(80):
        for q in range(D // 16):
            rows_a[r, pl.ds(q * 16, 16)] = jnp.zeros((16,), jnp.float32)
    rows_per_tile = N_PAD // NS
    base = s * rows_per_tile

    def zbody(b, carry):
        pltpu.sync_copy(rows_a.at[pl.ds(0, 80)], y_sh.at[pl.ds(base + b * 80, 80)])
        return carry

    lax.fori_loop(0, rows_per_tile // 80, zbody, 0)
    plsc.subcore_barrier()

    C2 = C // 2
    for h in range(2):
        pltpu.sync_copy(src_hbm.at[c, s, pl.ds(h * C2, C2)], src_v)
        pltpu.sync_copy(dst_hbm.at[c, s, pl.ds(h * C2, C2)], dst_v)
        pltpu.async_copy(t_hbm.at[src_v.at[0]], rows_a, sem_a)

        def ebody(i, carry):
            j = 2 * i
            pltpu.async_copy(t_hbm.at[src_v.at[j + 1]], rows_b, sem_b)
            pltpu.make_async_copy(t_hbm.at[src_v.at[j]], rows_a, sem_a).wait()
            pltpu.sync_copy(rows_a, y_sh.at[dst_v.at[j]], add=True)

            @pl.when(j + 2 < C2)
            def _():
                pltpu.async_copy(t_hbm.at[src_v.at[j + 2]], rows_a, sem_a)

            pltpu.make_async_copy(t_hbm.at[src_v.at[j + 1]], rows_b, sem_b).wait()
            pltpu.sync_copy(rows_b, y_sh.at[dst_v.at[j + 1]], add=True)
            return carry

        lax.fori_loop(0, C2 // 2, ebody, 0)
    plsc.subcore_barrier()
    pltpu.sync_copy(
        y_sh.at[pl.ds(base, rows_per_tile)],
        out_hbm.at[c, pl.ds(base, rows_per_tile)],
    )


@functools.partial(
    pl.kernel,
    out_type=[
        jax.ShapeDtypeStruct((NC, N_PAD, D), jnp.float32),
        jax.ShapeDtypeStruct((NC, N_PAD, D), jnp.float32),
    ],
    mesh=_mesh,
    scratch_types=[
        pltpu.VMEM((C // 2, K), jnp.int32),
        pltpu.VMEM((C // 2, K), jnp.int32),
        pltpu.VMEM((K, D), jnp.float32),
        pltpu.VMEM((K, D), jnp.float32),
        pltpu.VMEM_SHARED((N_PAD, D), jnp.float32),
        pltpu.SemaphoreType.DMA,
        pltpu.SemaphoreType.DMA,
    ],
)
def _sc_seg1(t_hbm, src_hbm, dst_hbm, y_out, deg_out, src_v, dst_v, rows_a,
             rows_b, y_sh, sem_a, sem_b):
    c = lax.axis_index("c")
    s = lax.axis_index("s")
    for r in range(80):
        for q in range(D // 16):
            rows_a[r, pl.ds(q * 16, 16)] = jnp.zeros((16,), jnp.float32)
    rows_per_tile = N_PAD // NS
    base = s * rows_per_tile

    def zbody(b, carry):
        pltpu.sync_copy(rows_a.at[pl.ds(0, 80)], y_sh.at[pl.ds(base + b * 80, 80)])
        return carry

    lax.fori_loop(0, rows_per_tile // 80, zbody, 0)
    plsc.subcore_barrier()

    C2 = C // 2
    for h in range(2):
        pltpu.sync_copy(src_hbm.at[c, s, pl.ds(h * C2, C2)], src_v)
        pltpu.sync_copy(dst_hbm.at[c, s, pl.ds(h * C2, C2)], dst_v)
        pltpu.async_copy(t_hbm.at[src_v.at[0]], rows_a, sem_a)

        def ebody(i, carry):
            j = 2 * i
            pltpu.async_copy(t_hbm.at[src_v.at[j + 1]], rows_b, sem_b)
            pltpu.make_async_copy(t_hbm.at[src_v.at[j]], rows_a, sem_a).wait()
            pltpu.sync_copy(rows_a, y_sh.at[dst_v.at[j]], add=True)

            @pl.when(j + 2 < C2)
            def _():
                pltpu.async_copy(t_hbm.at[src_v.at[j + 2]], rows_a, sem_a)

            pltpu.make_async_copy(t_hbm.at[src_v.at[j + 1]], rows_b, sem_b).wait()
            pltpu.sync_copy(rows_b, y_sh.at[dst_v.at[j + 1]], add=True)
            return carry

        lax.fori_loop(0, C2 // 2, ebody, 0)
    plsc.subcore_barrier()
    pltpu.sync_copy(
        y_sh.at[pl.ds(base, rows_per_tile)],
        y_out.at[c, pl.ds(base, rows_per_tile)],
    )
    for r in range(80):
        for q in range(D // 16):
            rows_b[r, pl.ds(q * 16, 16)] = jnp.zeros((16,), jnp.float32)

    def zbody2(b, carry):
        pltpu.sync_copy(rows_b.at[pl.ds(0, 80)], y_sh.at[pl.ds(base + b * 80, 80)])
        return carry

    lax.fori_loop(0, rows_per_tile // 80, zbody2, 0)
    for r in range(K):
        for q in range(D // 16):
            rows_a[r, pl.ds(q * 16, 16)] = jnp.ones((16,), jnp.float32)
    plsc.subcore_barrier()
    for h in range(2):
        pltpu.sync_copy(dst_hbm.at[c, s, pl.ds(h * C2, C2)], dst_v)

        def dbody(j, carry):
            pltpu.sync_copy(rows_a, y_sh.at[dst_v.at[j]], add=True)
            return carry

        lax.fori_loop(0, C2, dbody, 0)
    plsc.subcore_barrier()
    pltpu.sync_copy(
        y_sh.at[pl.ds(base, rows_per_tile)],
        deg_out.at[c, pl.ds(base, rows_per_tile)],
    )


DW = 128


@functools.partial(
    pl.kernel,
    out_type=jax.ShapeDtypeStruct((NC, N_PAD, DW), jnp.float32),
    mesh=_mesh,
    scratch_types=[
        pltpu.VMEM((C, K), jnp.int32),
        pltpu.VMEM((K, DW), jnp.float32),
        pltpu.VMEM((80, DW), jnp.float32),
        pltpu.VMEM_SHARED((N_PAD, DW), jnp.float32),
    ],
)
def _sc_deg(dst_hbm, out_hbm, dst_v, ones_v, zb, deg_sh):
    c = lax.axis_index("c")
    s = lax.axis_index("s")
    pltpu.sync_copy(dst_hbm.at[c, s], dst_v)
    for r in range(K):
        for q in range(DW // 16):
            ones_v[r, pl.ds(q * 16, 16)] = jnp.ones((16,), jnp.float32)
    for r in range(80):
        for q in range(DW // 16):
            zb[r, pl.ds(q * 16, 16)] = jnp.zeros((16,), jnp.float32)
    rows_per_tile = N_PAD // NS
    base = s * rows_per_tile

    def zbody(b, carry):
        pltpu.sync_copy(zb, deg_sh.at[pl.ds(base + b * 80, 80)])
        return carry

    lax.fori_loop(0, rows_per_tile // 80, zbody, 0)
    plsc.subcore_barrier()

    def ebody(j, carry):
        pltpu.sync_copy(ones_v, deg_sh.at[dst_v.at[j]], add=True)
        return carry

    lax.fori_loop(0, C, ebody, 0)
    plsc.subcore_barrier()
    pltpu.sync_copy(
        deg_sh.at[pl.ds(base, rows_per_tile)],
        out_hbm.at[c, pl.ds(base, rows_per_tile)],
    )


BN = 1024


def _mm_body(x_ref, w_ref, o_ref):
    o_ref[...] = jnp.dot(x_ref[...], w_ref[...], preferred_element_type=jnp.float32)


_tc_matmul = pl.pallas_call(
    _mm_body,
    grid=(N_PAD // BN,),
    in_specs=[
        pl.BlockSpec((BN, D), lambda i: (i, 0)),
        pl.BlockSpec((D, D), lambda i: (0, 0)),
    ],
    out_specs=pl.BlockSpec((BN, D), lambda i: (i, 0)),
    out_shape=jax.ShapeDtypeStruct((N_PAD, D), jnp.float32),
)


def _update_rows(x_ref, y_ref, dg_ref, wu1_ref, bu1_ref, wu2_ref, bu2_ref,
                 g_ref, b_ref):
    x = x_ref[...]
    y = y_ref[0] + y_ref[1]
    deg = jnp.maximum(dg_ref[0, :, 0:1] + dg_ref[1, :, 0:1], 1.0)
    agg = y / deg
    u = jnp.concatenate([x, agg], axis=1)
    h = jnp.maximum(
        jnp.dot(u, wu1_ref[...], preferred_element_type=jnp.float32) + bu1_ref[...],
        0.0,
    )
    upd = jnp.dot(h, wu2_ref[...], preferred_element_type=jnp.float32) + bu2_ref[...]
    mu = jnp.mean(upd, axis=-1, keepdims=True)
    var = jnp.mean((upd - mu) ** 2, axis=-1, keepdims=True)
    upd = (upd - mu) * lax.rsqrt(var + EPS) * g_ref[...] + b_ref[...]
    return x + upd


def _upd_body(x_ref, y_ref, dg_ref, wu1_ref, bu1_ref, wu2_ref, bu2_ref,
              g_ref, b_ref, wr_ref, xo_ref, to_ref):
    xn = _update_rows(x_ref, y_ref, dg_ref, wu1_ref, bu1_ref, wu2_ref, bu2_ref,
                      g_ref, b_ref)
    xo_ref[...] = xn
    to_ref[...] = jnp.dot(xn, wr_ref[...], preferred_element_type=jnp.float32)


def _upd_last_body(x_ref, y_ref, dg_ref, wu1_ref, bu1_ref, wu2_ref, bu2_ref,
                   g_ref, b_ref, xo_ref):
    xo_ref[...] = _update_rows(x_ref, y_ref, dg_ref, wu1_ref, bu1_ref, wu2_ref,
                               bu2_ref, g_ref, b_ref)


_upd_in_specs = [
    pl.BlockSpec((BN, D), lambda i: (i, 0)),
    pl.BlockSpec((NC, BN, D), lambda i: (0, i, 0)),
    pl.BlockSpec((NC, BN, DW), lambda i: (0, i, 0)),
    pl.BlockSpec((2 * D, D), lambda i: (0, 0)),
    pl.BlockSpec((1, D), lambda i: (0, 0)),
    pl.BlockSpec((D, D), lambda i: (0, 0)),
    pl.BlockSpec((1, D), lambda i: (0, 0)),
    pl.BlockSpec((1, D), lambda i: (0, 0)),
    pl.BlockSpec((1, D), lambda i: (0, 0)),
]

_tc_update = pl.pallas_call(
    _upd_body,
    grid=(N_PAD // BN,),
    in_specs=_upd_in_specs + [pl.BlockSpec((D, D), lambda i: (0, 0))],
    out_specs=[
        pl.BlockSpec((BN, D), lambda i: (i, 0)),
        pl.BlockSpec((BN, D), lambda i: (i, 0)),
    ],
    out_shape=[
        jax.ShapeDtypeStruct((N_PAD, D), jnp.float32),
        jax.ShapeDtypeStruct((N_PAD, D), jnp.float32),
    ],
)

_tc_update_last = pl.pallas_call(
    _upd_last_body,
    grid=(N_PAD // BN,),
    in_specs=_upd_in_specs,
    out_specs=pl.BlockSpec((BN, D), lambda i: (i, 0)),
    out_shape=jax.ShapeDtypeStruct((N_PAD, D), jnp.float32),
)


def kernel(node_embeddings, edge_index, W_rel, Wu1, bu1, Wu2, bu2, ln_g, ln_b):
    x = jnp.zeros((N_PAD, D), jnp.float32).at[:N].set(node_embeddings)
    src_r = edge_index[0].reshape(NC, NS, C, K)
    dst_r = edge_index[1].reshape(NC, NS, C, K)
    t = _tc_matmul(x, W_rel)
    b1 = bu1.reshape(1, D)
    b2 = bu2.reshape(1, D)
    g = ln_g.reshape(1, D)
    b = ln_b.reshape(1, D)
    y, degp = _sc_seg1(t, src_r, dst_r)
    x, t = _tc_update(x, y, degp, Wu1, b1, Wu2, b2, g, b, W_rel)
    for _ in range(NUM_LAYERS - 2):
        y = _sc_segsum(t, src_r, dst_r)
        x, t = _tc_update(x, y, degp, Wu1, b1, Wu2, b2, g, b, W_rel)
    y = _sc_segsum(t, src_r, dst_r)
    x = _tc_update_last(x, y, degp, Wu1, b1, Wu2, b2, g, b)
    return x[:N]


# speedup vs baseline: 11.1703x; 1.0133x over previous
import functools

import jax
import jax.numpy as jnp
from jax import lax
from jax.experimental import pallas as pl
from jax.experimental.pallas import tpu as pltpu
from jax.experimental.pallas import tpu_sc as plsc

N = 10000
E = 320000
D = 128
NUM_LAYERS = 3
EPS = 1e-5

NC = 2
NS = 16
NW = NC * NS
N_PAD = 10240
K = 125
C = E // (NW * K)

_mesh = plsc.VectorSubcoreMesh(
    core_axis_name="c", subcore_axis_name="s", num_cores=NC, num_subcores=NS
)


@functools.partial(
    pl.kernel,
    out_type=jax.ShapeDtypeStruct((NC, N_PAD, D), jnp.float32),
    mesh=_mesh,
    scratch_types=[
        pltpu.VMEM((C // 2, K), jnp.int32),
        pltpu.VMEM((C // 2, K), jnp.int32),
        pltpu.VMEM((K, D), jnp.float32),
        pltpu.VMEM((K, D), jnp.float32),
        pltpu.VMEM_SHARED((N_PAD, D), jnp.float32),
        pltpu.SemaphoreType.DMA,
        pltpu.SemaphoreType.DMA,
    ],
)
def _sc_segsum(t_hbm, src_hbm, dst_hbm, out_hbm, src_v, dst_v, rows_a, rows_b,
               y_sh, sem_a, sem_b):
    c = lax.axis_index("c")
    s = lax.axis_index("s")
    for r in range(80):
        for q in range(D // 16):
            rows_a[r, pl.ds(q * 16, 16)] = jnp.zeros((16,), jnp.float32)
    rows_per_tile = N_PAD // NS
    base = s * rows_per_tile

    def zbody(b, carry):
        pltpu.sync_copy(rows_a.at[pl.ds(0, 80)], y_sh.at[pl.ds(base + b * 80, 80)])
        return carry

    lax.fori_loop(0, rows_per_tile // 80, zbody, 0)
    plsc.subcore_barrier()

    C2 = C // 2
    for h in range(2):
        pltpu.sync_copy(src_hbm.at[c, s, pl.ds(h * C2, C2)], src_v)
        pltpu.sync_copy(dst_hbm.at[c, s, pl.ds(h * C2, C2)], dst_v)
        pltpu.async_copy(t_hbm.at[src_v.at[0]], rows_a, sem_a)

        def ebody(i, carry):
            j = 2 * i
            pltpu.async_copy(t_hbm.at[src_v.at[j + 1]], rows_b, sem_b)
            pltpu.make_async_copy(t_hbm.at[src_v.at[j]], rows_a, sem_a).wait()
            pltpu.sync_copy(rows_a, y_sh.at[dst_v.at[j]], add=True)

            @pl.when(j + 2 < C2)
            def _():
                pltpu.async_copy(t_hbm.at[src_v.at[j + 2]], rows_a, sem_a)

            pltpu.make_async_copy(t_hbm.at[src_v.at[j + 1]], rows_b, sem_b).wait()
            pltpu.sync_copy(rows_b, y_sh.at[dst_v.at[j + 1]], add=True)
            return carry

        lax.fori_loop(0, C2 // 2, ebody, 0)
    plsc.subcore_barrier()
    pltpu.sync_copy(
        y_sh.at[pl.ds(base, rows_per_tile)],
        out_hbm.at[c, pl.ds(base, rows_per_tile)],
    )


@functools.partial(
    pl.kernel,
    out_type=[
        jax.ShapeDtypeStruct((NC, N_PAD, D), jnp.float32),
        jax.ShapeDtypeStruct((NC, N_PAD, D), jnp.float32),
    ],
    mesh=_mesh,
    scratch_types=[
        pltpu.VMEM((C // 2, K), jnp.int32),
        pltpu.VMEM((C // 2, K), jnp.int32),
        pltpu.VMEM((K, D), jnp.float32),
        pltpu.VMEM((K, D), jnp.float32),
        pltpu.VMEM_SHARED((N_PAD, D), jnp.float32),
        pltpu.SemaphoreType.DMA,
        pltpu.SemaphoreType.DMA,
    ],
)
def _sc_seg1(t_hbm, src_hbm, dst_hbm, y_out, deg_out, src_v, dst_v, rows_a,
             rows_b, y_sh, sem_a, sem_b):
    c = lax.axis_index("c")
    s = lax.axis_index("s")
    for r in range(80):
        for q in range(D // 16):
            rows_a[r, pl.ds(q * 16, 16)] = jnp.zeros((16,), jnp.float32)
    rows_per_tile = N_PAD // NS
    base = s * rows_per_tile

    def zbody(b, carry):
        pltpu.sync_copy(rows_a.at[pl.ds(0, 80)], y_sh.at[pl.ds(base + b * 80, 80)])
        return carry

    lax.fori_loop(0, rows_per_tile // 80, zbody, 0)
    plsc.subcore_barrier()

    C2 = C // 2
    for h in range(2):
        pltpu.sync_copy(src_hbm.at[c, s, pl.ds(h * C2, C2)], src_v)
        pltpu.sync_copy(dst_hbm.at[c, s, pl.ds(h * C2, C2)], dst_v)
        pltpu.async_copy(t_hbm.at[src_v.at[0]], rows_a, sem_a)

        def ebody(i, carry):
            j = 2 * i
            pltpu.async_copy(t_hbm.at[src_v.at[j + 1]], rows_b, sem_b)
            pltpu.make_async_copy(t_hbm.at[src_v.at[j]], rows_a, sem_a).wait()
            pltpu.sync_copy(rows_a, y_sh.at[dst_v.at[j]], add=True)

            @pl.when(j + 2 < C2)
            def _():
                pltpu.async_copy(t_hbm.at[src_v.at[j + 2]], rows_a, sem_a)

            pltpu.make_async_copy(t_hbm.at[src_v.at[j + 1]], rows_b, sem_b).wait()
            pltpu.sync_copy(rows_b, y_sh.at[dst_v.at[j + 1]], add=True)
            return carry

        lax.fori_loop(0, C2 // 2, ebody, 0)
    plsc.subcore_barrier()
    pltpu.sync_copy(
        y_sh.at[pl.ds(base, rows_per_tile)],
        y_out.at[c, pl.ds(base, rows_per_tile)],
    )
    for r in range(80):
        for q in range(D // 16):
            rows_b[r, pl.ds(q * 16, 16)] = jnp.zeros((16,), jnp.float32)

    def zbody2(b, carry):
        pltpu.sync_copy(rows_b.at[pl.ds(0, 80)], y_sh.at[pl.ds(base + b * 80, 80)])
        return carry

    lax.fori_loop(0, rows_per_tile // 80, zbody2, 0)
    for r in range(K):
        for q in range(D // 16):
            rows_a[r, pl.ds(q * 16, 16)] = jnp.ones((16,), jnp.float32)
    plsc.subcore_barrier()
    for h in range(2):
        pltpu.sync_copy(dst_hbm.at[c, s, pl.ds(h * C2, C2)], dst_v)

        def dbody(j, carry):
            pltpu.sync_copy(rows_a, y_sh.at[dst_v.at[j]], add=True)
            return carry

        lax.fori_loop(0, C2, dbody, 0)
    plsc.subcore_barrier()
    pltpu.sync_copy(
        y_sh.at[pl.ds(base, rows_per_tile)],
        deg_out.at[c, pl.ds(base, rows_per_tile)],
    )


DW = 128


@functools.partial(
    pl.kernel,
    out_type=jax.ShapeDtypeStruct((NC, N_PAD, DW), jnp.float32),
    mesh=_mesh,
    scratch_types=[
        pltpu.VMEM((C, K), jnp.int32),
        pltpu.VMEM((K, DW), jnp.float32),
        pltpu.VMEM((80, DW), jnp.float32),
        pltpu.VMEM_SHARED((N_PAD, DW), jnp.float32),
    ],
)
def _sc_deg(dst_hbm, out_hbm, dst_v, ones_v, zb, deg_sh):
    c = lax.axis_index("c")
    s = lax.axis_index("s")
    pltpu.sync_copy(dst_hbm.at[c, s], dst_v)
    for r in range(K):
        for q in range(DW // 16):
            ones_v[r, pl.ds(q * 16, 16)] = jnp.ones((16,), jnp.float32)
    for r in range(80):
        for q in range(DW // 16):
            zb[r, pl.ds(q * 16, 16)] = jnp.zeros((16,), jnp.float32)
    rows_per_tile = N_PAD // NS
    base = s * rows_per_tile

    def zbody(b, carry):
        pltpu.sync_copy(zb, deg_sh.at[pl.ds(base + b * 80, 80)])
        return carry

    lax.fori_loop(0, rows_per_tile // 80, zbody, 0)
    plsc.subcore_barrier()

    def ebody(j, carry):
        pltpu.sync_copy(ones_v, deg_sh.at[dst_v.at[j]], add=True)
        return carry

    lax.fori_loop(0, C, ebody, 0)
    plsc.subcore_barrier()
    pltpu.sync_copy(
        deg_sh.at[pl.ds(base, rows_per_tile)],
        out_hbm.at[c, pl.ds(base, rows_per_tile)],
    )


BN = 1000


def _mm_body(x_ref, w_ref, o_ref):
    o_ref[...] = jnp.dot(x_ref[...], w_ref[...], preferred_element_type=jnp.float32)


_tc_matmul = pl.pallas_call(
    _mm_body,
    grid=(N // BN,),
    in_specs=[
        pl.BlockSpec((BN, D), lambda i: (i, 0)),
        pl.BlockSpec((D, D), lambda i: (0, 0)),
    ],
    out_specs=pl.BlockSpec((BN, D), lambda i: (i, 0)),
    out_shape=jax.ShapeDtypeStruct((N, D), jnp.float32),
)


def _update_rows(x_ref, y_ref, dg_ref, wu1_ref, bu1_ref, wu2_ref, bu2_ref,
                 g_ref, b_ref):
    x = x_ref[...]
    y = y_ref[0] + y_ref[1]
    deg = jnp.maximum(dg_ref[0, :, 0:1] + dg_ref[1, :, 0:1], 1.0)
    agg = y / deg
    u = jnp.concatenate([x, agg], axis=1)
    h = jnp.maximum(
        jnp.dot(u, wu1_ref[...], preferred_element_type=jnp.float32) + bu1_ref[...],
        0.0,
    )
    upd = jnp.dot(h, wu2_ref[...], preferred_element_type=jnp.float32) + bu2_ref[...]
    mu = jnp.mean(upd, axis=-1, keepdims=True)
    var = jnp.mean((upd - mu) ** 2, axis=-1, keepdims=True)
    upd = (upd - mu) * lax.rsqrt(var + EPS) * g_ref[...] + b_ref[...]
    return x + upd


def _upd_body(x_ref, y_ref, dg_ref, wu1_ref, bu1_ref, wu2_ref, bu2_ref,
              g_ref, b_ref, wr_ref, xo_ref, to_ref):
    xn = _update_rows(x_ref, y_ref, dg_ref, wu1_ref, bu1_ref, wu2_ref, bu2_ref,
                      g_ref, b_ref)
    xo_ref[...] = xn
    to_ref[...] = jnp.dot(xn, wr_ref[...], preferred_element_type=jnp.float32)


def _upd_last_body(x_ref, y_ref, dg_ref, wu1_ref, bu1_ref, wu2_ref, bu2_ref,
                   g_ref, b_ref, xo_ref):
    xo_ref[...] = _update_rows(x_ref, y_ref, dg_ref, wu1_ref, bu1_ref, wu2_ref,
                               bu2_ref, g_ref, b_ref)


_upd_in_specs = [
    pl.BlockSpec((BN, D), lambda i: (i, 0)),
    pl.BlockSpec((NC, BN, D), lambda i: (0, i, 0)),
    pl.BlockSpec((NC, BN, DW), lambda i: (0, i, 0)),
    pl.BlockSpec((2 * D, D), lambda i: (0, 0)),
    pl.BlockSpec((1, D), lambda i: (0, 0)),
    pl.BlockSpec((D, D), lambda i: (0, 0)),
    pl.BlockSpec((1, D), lambda i: (0, 0)),
    pl.BlockSpec((1, D), lambda i: (0, 0)),
    pl.BlockSpec((1, D), lambda i: (0, 0)),
]

_tc_update = pl.pallas_call(
    _upd_body,
    grid=(N // BN,),
    in_specs=_upd_in_specs + [pl.BlockSpec((D, D), lambda i: (0, 0))],
    out_specs=[
        pl.BlockSpec((BN, D), lambda i: (i, 0)),
        pl.BlockSpec((BN, D), lambda i: (i, 0)),
    ],
    out_shape=[
        jax.ShapeDtypeStruct((N, D), jnp.float32),
        jax.ShapeDtypeStruct((N, D), jnp.float32),
    ],
)

_tc_update_last = pl.pallas_call(
    _upd_last_body,
    grid=(N // BN,),
    in_specs=_upd_in_specs,
    out_specs=pl.BlockSpec((BN, D), lambda i: (i, 0)),
    out_shape=jax.ShapeDtypeStruct((N, D), jnp.float32),
)


def kernel(node_embeddings, edge_index, W_rel, Wu1, bu1, Wu2, bu2, ln_g, ln_b):
    x = node_embeddings
    src_r = edge_index[0].reshape(NC, NS, C, K)
    dst_r = edge_index[1].reshape(NC, NS, C, K)
    t = _tc_matmul(x, W_rel)
    b1 = bu1.reshape(1, D)
    b2 = bu2.reshape(1, D)
    g = ln_g.reshape(1, D)
    b = ln_b.reshape(1, D)
    y, degp = _sc_seg1(t, src_r, dst_r)
    x, t = _tc_update(x, y, degp, Wu1, b1, Wu2, b2, g, b, W_rel)
    for _ in range(NUM_LAYERS - 2):
        y = _sc_segsum(t, src_r, dst_r)
        x, t = _tc_update(x, y, degp, Wu1, b1, Wu2, b2, g, b, W_rel)
    y = _sc_segsum(t, src_r, dst_r)
    return _tc_update_last(x, y, degp, Wu1, b1, Wu2, b2, g, b)


# speedup vs baseline: 11.1759x; 1.0005x over previous
import functools

import jax
import jax.numpy as jnp
from jax import lax
from jax.experimental import pallas as pl
from jax.experimental.pallas import tpu as pltpu
from jax.experimental.pallas import tpu_sc as plsc

N = 10000
E = 320000
D = 128
NUM_LAYERS = 3
EPS = 1e-5

NC = 2
NS = 16
NW = NC * NS
N_PAD = 10240
K = 125
C = E // (NW * K)

_mesh = plsc.VectorSubcoreMesh(
    core_axis_name="c", subcore_axis_name="s", num_cores=NC, num_subcores=NS
)


@functools.partial(
    pl.kernel,
    out_type=jax.ShapeDtypeStruct((NC, N_PAD, D), jnp.float32),
    mesh=_mesh,
    scratch_types=[
        pltpu.VMEM((C // 2, K), jnp.int32),
        pltpu.VMEM((C // 2, K), jnp.int32),
        pltpu.VMEM((K, D), jnp.float32),
        pltpu.VMEM((K, D), jnp.float32),
        pltpu.VMEM_SHARED((N_PAD, D), jnp.float32),
        pltpu.SemaphoreType.DMA,
        pltpu.SemaphoreType.DMA,
    ],
)
def _sc_segsum(t_hbm, src_hbm, dst_hbm, out_hbm, src_v, dst_v, rows_a, rows_b,
               y_sh, sem_a, sem_b):
    c = lax.axis_index("c")
    s = lax.axis_index("s")
    for r in range(80):
        for q in range(D // 16):
            rows_a[r, pl.ds(q * 16, 16)] = jnp.zeros((16,), jnp.float32)
    rows_per_tile = N_PAD // NS
    base = s * rows_per_tile

    def zbody(b, carry):
        pltpu.sync_copy(rows_a.at[pl.ds(0, 80)], y_sh.at[pl.ds(base + b * 80, 80)])
        return carry

    lax.fori_loop(0, rows_per_tile // 80, zbody, 0)
    plsc.subcore_barrier()

    C2 = C // 2
    for h in range(2):
        pltpu.sync_copy(src_hbm.at[c, s, pl.ds(h * C2, C2)], src_v)
        pltpu.sync_copy(dst_hbm.at[c, s, pl.ds(h * C2, C2)], dst_v)
        pltpu.async_copy(t_hbm.at[src_v.at[0]], rows_a, sem_a)

        def ebody(i, carry):
            j = 2 * i
            pltpu.async_copy(t_hbm.at[src_v.at[j + 1]], rows_b, sem_b)
            pltpu.make_async_copy(t_hbm.at[src_v.at[j]], rows_a, sem_a).wait()
            pltpu.sync_copy(rows_a, y_sh.at[dst_v.at[j]], add=True)

            @pl.when(j + 2 < C2)
            def _():
                pltpu.async_copy(t_hbm.at[src_v.at[j + 2]], rows_a, sem_a)

            pltpu.make_async_copy(t_hbm.at[src_v.at[j + 1]], rows_b, sem_b).wait()
            pltpu.sync_copy(rows_b, y_sh.at[dst_v.at[j + 1]], add=True)
            return carry

        lax.fori_loop(0, C2 // 2, ebody, 0)
    plsc.subcore_barrier()
    pltpu.sync_copy(
        y_sh.at[pl.ds(base, rows_per_tile)],
        out_hbm.at[c, pl.ds(base, rows_per_tile)],
    )


@functools.partial(
    pl.kernel,
    out_type=[
        jax.ShapeDtypeStruct((NC, N_PAD, D), jnp.float32),
        jax.ShapeDtypeStruct((NC, N_PAD, D), jnp.float32),
    ],
    mesh=_mesh,
    scratch_types=[
        pltpu.VMEM((C // 2, K), jnp.int32),
        pltpu.VMEM((C // 2, K), jnp.int32),
        pltpu.VMEM((K, D), jnp.float32),
        pltpu.VMEM((K, D), jnp.float32),
        pltpu.VMEM_SHARED((N_PAD, D), jnp.float32),
        pltpu.SemaphoreType.DMA,
        pltpu.SemaphoreType.DMA,
    ],
)
def _sc_seg1(t_hbm, src_hbm, dst_hbm, y_out, deg_out, src_v, dst_v, rows_a,
             rows_b, y_sh, sem_a, sem_b):
    c = lax.axis_index("c")
    s = lax.axis_index("s")
    for r in range(80):
        for q in range(D // 16):
            rows_a[r, pl.ds(q * 16, 16)] = jnp.zeros((16,), jnp.float32)
    rows_per_tile = N_PAD // NS
    base = s * rows_per_tile

    def zbody(b, carry):
        pltpu.sync_copy(rows_a.at[pl.ds(0, 80)], y_sh.at[pl.ds(base + b * 80, 80)])
        return carry

    lax.fori_loop(0, rows_per_tile // 80, zbody, 0)
    plsc.subcore_barrier()

    C2 = C // 2
    for h in range(2):
        pltpu.sync_copy(src_hbm.at[c, s, pl.ds(h * C2, C2)], src_v)
        pltpu.sync_copy(dst_hbm.at[c, s, pl.ds(h * C2, C2)], dst_v)
        pltpu.async_copy(t_hbm.at[src_v.at[0]], rows_a, sem_a)

        def ebody(i, carry):
            j = 2 * i
            pltpu.async_copy(t_hbm.at[src_v.at[j + 1]], rows_b, sem_b)
            pltpu.make_async_copy(t_hbm.at[src_v.at[j]], rows_a, sem_a).wait()
            pltpu.sync_copy(rows_a, y_sh.at[dst_v.at[j]], add=True)

            @pl.when(j + 2 < C2)
            def _():
                pltpu.async_copy(t_hbm.at[src_v.at[j + 2]], rows_a, sem_a)

            pltpu.make_async_copy(t_hbm.at[src_v.at[j + 1]], rows_b, sem_b).wait()
            pltpu.sync_copy(rows_b, y_sh.at[dst_v.at[j + 1]], add=True)
            return carry

        lax.fori_loop(0, C2 // 2, ebody, 0)
    plsc.subcore_barrier()
    pltpu.sync_copy(
        y_sh.at[pl.ds(base, rows_per_tile)],
        y_out.at[c, pl.ds(base, rows_per_tile)],
    )
    for r in range(80):
        for q in range(D // 16):
            rows_b[r, pl.ds(q * 16, 16)] = jnp.zeros((16,), jnp.float32)

    def zbody2(b, carry):
        pltpu.sync_copy(rows_b.at[pl.ds(0, 80)], y_sh.at[pl.ds(base + b * 80, 80)])
        return carry

    lax.fori_loop(0, rows_per_tile // 80, zbody2, 0)
    for r in range(K):
        for q in range(D // 16):
            rows_a[r, pl.ds(q * 16, 16)] = jnp.ones((16,), jnp.float32)
    plsc.subcore_barrier()
    for h in range(2):
        pltpu.sync_copy(dst_hbm.at[c, s, pl.ds(h * C2, C2)], dst_v)

        def dbody(j, carry):
            pltpu.sync_copy(rows_a, y_sh.at[dst_v.at[j]], add=True)
            return carry

        lax.fori_loop(0, C2, dbody, 0)
    plsc.subcore_barrier()
    pltpu.sync_copy(
        y_sh.at[pl.ds(base, rows_per_tile)],
        deg_out.at[c, pl.ds(base, rows_per_tile)],
    )


BN = 1000


def _mm_body(x_ref, w_ref, o_ref):
    o_ref[...] = jnp.dot(x_ref[...], w_ref[...], preferred_element_type=jnp.float32)


_tc_matmul = pl.pallas_call(
    _mm_body,
    grid=(N // BN,),
    in_specs=[
        pl.BlockSpec((BN, D), lambda i: (i, 0)),
        pl.BlockSpec((D, D), lambda i: (0, 0)),
    ],
    out_specs=pl.BlockSpec((BN, D), lambda i: (i, 0)),
    out_shape=jax.ShapeDtypeStruct((N, D), jnp.float32),
)


def _update_rows(x_ref, y_ref, dg_ref, wu1_ref, bu1_ref, wu2_ref, bu2_ref,
                 g_ref, b_ref):
    x = x_ref[...]
    y = y_ref[0] + y_ref[1]
    deg = jnp.maximum(dg_ref[0, :, 0:1] + dg_ref[1, :, 0:1], 1.0)
    agg = y / deg
    u = jnp.concatenate([x, agg], axis=1)
    h = jnp.maximum(
        jnp.dot(u, wu1_ref[...], preferred_element_type=jnp.float32) + bu1_ref[...],
        0.0,
    )
    upd = jnp.dot(h, wu2_ref[...], preferred_element_type=jnp.float32) + bu2_ref[...]
    mu = jnp.mean(upd, axis=-1, keepdims=True)
    var = jnp.mean((upd - mu) ** 2, axis=-1, keepdims=True)
    upd = (upd - mu) * lax.rsqrt(var + EPS) * g_ref[...] + b_ref[...]
    return x + upd


def _upd_body(x_ref, y_ref, dg_ref, wu1_ref, bu1_ref, wu2_ref, bu2_ref,
              g_ref, b_ref, wr_ref, xo_ref, to_ref):
    xn = _update_rows(x_ref, y_ref, dg_ref, wu1_ref, bu1_ref, wu2_ref, bu2_ref,
                      g_ref, b_ref)
    xo_ref[...] = xn
    to_ref[...] = jnp.dot(xn, wr_ref[...], preferred_element_type=jnp.float32)


def _upd_last_body(x_ref, y_ref, dg_ref, wu1_ref, bu1_ref, wu2_ref, bu2_ref,
                   g_ref, b_ref, xo_ref):
    xo_ref[...] = _update_rows(x_ref, y_ref, dg_ref, wu1_ref, bu1_ref, wu2_ref,
                               bu2_ref, g_ref, b_ref)


_upd_in_specs = [
    pl.BlockSpec((BN, D), lambda i: (i, 0)),
    pl.BlockSpec((NC, BN, D), lambda i: (0, i, 0)),
    pl.BlockSpec((NC, BN, D), lambda i: (0, i, 0)),
    pl.BlockSpec((2 * D, D), lambda i: (0, 0)),
    pl.BlockSpec((1, D), lambda i: (0, 0)),
    pl.BlockSpec((D, D), lambda i: (0, 0)),
    pl.BlockSpec((1, D), lambda i: (0, 0)),
    pl.BlockSpec((1, D), lambda i: (0, 0)),
    pl.BlockSpec((1, D), lambda i: (0, 0)),
]

_tc_update = pl.pallas_call(
    _upd_body,
    grid=(N // BN,),
    in_specs=_upd_in_specs + [pl.BlockSpec((D, D), lambda i: (0, 0))],
    out_specs=[
        pl.BlockSpec((BN, D), lambda i: (i, 0)),
        pl.BlockSpec((BN, D), lambda i: (i, 0)),
    ],
    out_shape=[
        jax.ShapeDtypeStruct((N, D), jnp.float32),
        jax.ShapeDtypeStruct((N, D), jnp.float32),
    ],
)

_tc_update_last = pl.pallas_call(
    _upd_last_body,
    grid=(N // BN,),
    in_specs=_upd_in_specs,
    out_specs=pl.BlockSpec((BN, D), lambda i: (i, 0)),
    out_shape=jax.ShapeDtypeStruct((N, D), jnp.float32),
)


def kernel(node_embeddings, edge_index, W_rel, Wu1, bu1, Wu2, bu2, ln_g, ln_b):
    x = node_embeddings
    src_r = edge_index[0].reshape(NC, NS, C, K)
    dst_r = edge_index[1].reshape(NC, NS, C, K)
    t = _tc_matmul(x, W_rel)
    b1 = bu1.reshape(1, D)
    b2 = bu2.reshape(1, D)
    g = ln_g.reshape(1, D)
    b = ln_b.reshape(1, D)
    y, degp = _sc_seg1(t, src_r, dst_r)
    x, t = _tc_update(x, y, degp, Wu1, b1, Wu2, b2, g, b, W_rel)
    for _ in range(NUM_LAYERS - 2):
        y = _sc_segsum(t, src_r, dst_r)
        x, t = _tc_update(x, y, degp, Wu1, b1, Wu2, b2, g, b, W_rel)
    y = _sc_segsum(t, src_r, dst_r)
    return _tc_update_last(x, y, degp, Wu1, b1, Wu2, b2, g, b)
